# Initial kernel scaffold; baseline (speedup 1.0000x reference)
import jax
import jax.numpy as jnp
from jax.experimental import pallas as pl


def kernel(x, edge_index, W1, a_src1, a_dst1, b1, W2, a_src2, a_dst2, b2, W_ih_f, W_hh_f, b_ih_f, b_hh_f, W_ih_b, W_hh_b, b_ih_b, b_hh_b, W_att, b_att, W_out):
    raise NotImplementedError("write your pallas kernel here")


# speedup vs baseline: 43.7815x; 43.7815x over previous
import functools

import jax
import jax.numpy as jnp
from jax import lax
from jax.experimental import pallas as pl
from jax.experimental.pallas import tpu as pltpu
from jax.experimental.pallas import tpu_sc as plsc

N = 10000
E = 320000
HEADS = 8
OUT = 8
HID = 64
NUM_CLASSES = 40
F_IN = 128

NP = 10240
C = 128
ITERS = 81
NW = 32
EP = NW * ITERS * C
ETOT = E + N
ROWS_PER_SUB = NP // 16

_f32 = jnp.float32
_i32 = jnp.int32


def _mesh():
  return plsc.VectorSubcoreMesh(
      core_axis_name="c", subcore_axis_name="s", num_cores=2, num_subcores=16)


def _sc_pass1_body(src_hbm, dst_hbm, tas_hbm, tad_hbm,
                   p_out, ssum_out,
                   sidx, didx, a_buf, b_buf, pp_buf, pd_buf, accum):
  c = lax.axis_index("c")
  s = lax.axis_index("s")
  wid = s * 2 + c
  lane = lax.iota(_i32, 16)
  lo = lane < 8
  vz = jnp.zeros((16,), _f32)

  for r in range(C):
    pd_buf[r] = vz
  row0 = s * ROWS_PER_SUB
  for k in range(ROWS_PER_SUB // C):
    pltpu.sync_copy(pd_buf, accum.at[pl.ds(row0 + k * C, C)])
  plsc.subcore_barrier()

  ebase0 = wid * (ITERS * C)

  def chunk(it, _):
    base = pl.multiple_of(ebase0 + it * C, 8)
    pbase = pl.multiple_of(ebase0 // 2 + it * (C // 2), 8)
    pltpu.sync_copy(src_hbm.at[pl.ds(base, C)], sidx)
    pltpu.sync_copy(dst_hbm.at[pl.ds(base, C)], didx)
    pltpu.sync_copy(tas_hbm.at[sidx], a_buf)
    pltpu.sync_copy(tad_hbm.at[didx], b_buf)
    for j in range(C // 2):
      a0 = a_buf[2 * j]
      a1 = a_buf[2 * j + 1]
      b0 = b_buf[2 * j]
      b1 = b_buf[2 * j + 1]
      al = jnp.where(lo, a0, a1) + jnp.where(lo, b0, b1)
      al = jnp.maximum(al, 0.2 * al)
      p = jnp.exp(al)
      pp_buf[j] = p
    idx_lo = jnp.where(lo, lane, lane - 8)
    idx_hi = idx_lo + 8
    for j in range(C // 2):
      row = jnp.full((16,), j, _i32)
      pd_buf[2 * j] = plsc.load_gather(pp_buf, [row, idx_lo])
      pd_buf[2 * j + 1] = plsc.load_gather(pp_buf, [row, idx_hi])
    pltpu.sync_copy(pp_buf, p_out.at[pl.ds(pbase, C // 2)])
    pltpu.sync_copy(pd_buf, accum.at[didx], add=True)
    return _

  lax.fori_loop(0, ITERS, chunk, 0)

  plsc.subcore_barrier()
  pltpu.sync_copy(accum.at[pl.ds(row0, ROWS_PER_SUB)],
                  ssum_out.at[c, pl.ds(row0, ROWS_PER_SUB)])


def _sc_pass1(src, dst, tas, tad):
  f = pl.kernel(
      _sc_pass1_body,
      out_type=[
          jax.ShapeDtypeStruct((EP // 2, 16), _f32),
          jax.ShapeDtypeStruct((2, NP, 16), _f32),
      ],
      mesh=_mesh(),
      compiler_params=pltpu.CompilerParams(needs_layout_passes=False, use_tc_tiling_on_sc=False),
      scratch_types=[
          pltpu.VMEM((C,), _i32),
          pltpu.VMEM((C,), _i32),
          pltpu.VMEM((C, 16), _f32),
          pltpu.VMEM((C, 16), _f32),
          pltpu.VMEM((C // 2, 16), _f32),
          pltpu.VMEM((C, 16), _f32),
          pltpu.VMEM_SHARED((NP, 16), _f32),
      ],
  )
  return f(src, dst, tas, tad)


def _sc_pass2_body(src_hbm, dst_hbm, p_hbm, inv_hbm, xl_hbm,
                   msg_out,
                   sidx, didx, x_buf, i_buf, pp_buf, al_buf, m_buf, accum):
  c = lax.axis_index("c")
  s = lax.axis_index("s")
  wid = s * 2 + c
  lane = lax.iota(_i32, 16)
  lo = lane < 8
  vz = jnp.zeros((16,), _f32)

  for r in range(C):
    for v in range(4):
      m_buf[r, pl.ds(16 * v, 16)] = vz
  row0 = s * ROWS_PER_SUB
  for k in range(ROWS_PER_SUB // C):
    pltpu.sync_copy(m_buf, accum.at[pl.ds(row0 + k * C, C)])
  plsc.subcore_barrier()

  ebase0 = wid * (ITERS * C)
  col_idx = []
  for w in range(8):
    h0 = 2 * (w % 4)
    base = 8 * (w // 4)
    col_idx.append(jnp.where(lo, jnp.full((16,), base + h0, _i32),
                             jnp.full((16,), base + h0 + 1, _i32)))

  def chunk(it, _):
    base = pl.multiple_of(ebase0 + it * C, 8)
    pbase = pl.multiple_of(ebase0 // 2 + it * (C // 2), 8)
    pltpu.sync_copy(src_hbm.at[pl.ds(base, C)], sidx)
    pltpu.sync_copy(dst_hbm.at[pl.ds(base, C)], didx)
    pltpu.sync_copy(xl_hbm.at[sidx], x_buf)
    pltpu.sync_copy(inv_hbm.at[didx], i_buf)
    pltpu.sync_copy(p_hbm.at[pl.ds(pbase, C // 2)], pp_buf)
    for j in range(C // 2):
      i0 = i_buf[2 * j]
      i1 = i_buf[2 * j + 1]
      al_buf[j] = pp_buf[j] * jnp.where(lo, i0, i1)
    for j in range(C // 2):
      row = jnp.full((16,), j, _i32)
      for w in range(8):
        av = plsc.load_gather(al_buf, [row, col_idx[w]])
        e = 2 * j + (w // 4)
        v = w % 4
        m_buf[e, pl.ds(16 * v, 16)] = av * x_buf[e, pl.ds(16 * v, 16)]
    pltpu.sync_copy(m_buf, accum.at[didx], add=True)
    return _

  lax.fori_loop(0, ITERS, chunk, 0)

  plsc.subcore_barrier()
  pltpu.sync_copy(accum.at[pl.ds(row0, ROWS_PER_SUB)],
                  msg_out.at[c, pl.ds(row0, ROWS_PER_SUB)])


def _sc_pass2(src, dst, p, inv16, xl):
  f = pl.kernel(
      _sc_pass2_body,
      out_type=[jax.ShapeDtypeStruct((2, NP, HEADS * OUT), _f32)],
      mesh=_mesh(),
      compiler_params=pltpu.CompilerParams(needs_layout_passes=False, use_tc_tiling_on_sc=False),
      scratch_types=[
          pltpu.VMEM((C,), _i32),
          pltpu.VMEM((C,), _i32),
          pltpu.VMEM((C, HEADS * OUT), _f32),
          pltpu.VMEM((C, 16), _f32),
          pltpu.VMEM((C // 2, 16), _f32),
          pltpu.VMEM((C // 2, 16), _f32),
          pltpu.VMEM((C, HEADS * OUT), _f32),
          pltpu.VMEM_SHARED((NP, HEADS * OUT), _f32),
      ],
  )
  return f(src, dst, p, inv16, xl)[0]


_BLK = 1024


def _proj_body(x_ref, w_ref, ms_ref, md_ref, xl_ref, tas_ref, tad_ref):
  xl = jnp.dot(x_ref[...], w_ref[...], preferred_element_type=_f32)
  xl_ref[...] = xl
  tas_ref[...] = jnp.dot(xl, ms_ref[...], preferred_element_type=_f32)
  tad_ref[...] = jnp.dot(xl, md_ref[...], preferred_element_type=_f32)


def _tc_proj(x, w, ms16, md16):
  fin = x.shape[1]
  grid = (NP // _BLK,)
  return pl.pallas_call(
      _proj_body,
      grid=grid,
      in_specs=[
          pl.BlockSpec((_BLK, fin), lambda i: (i, 0)),
          pl.BlockSpec((fin, HEADS * OUT), lambda i: (0, 0)),
          pl.BlockSpec((HEADS * OUT, 16), lambda i: (0, 0)),
          pl.BlockSpec((HEADS * OUT, 16), lambda i: (0, 0)),
      ],
      out_specs=[
          pl.BlockSpec((_BLK, HEADS * OUT), lambda i: (i, 0)),
          pl.BlockSpec((_BLK, 16), lambda i: (i, 0)),
          pl.BlockSpec((_BLK, 16), lambda i: (i, 0)),
      ],
      out_shape=[
          jax.ShapeDtypeStruct((NP, HEADS * OUT), _f32),
          jax.ShapeDtypeStruct((NP, 16), _f32),
          jax.ShapeDtypeStruct((NP, 16), _f32),
      ],
  )(x, w, ms16, md16)


def _inv_body(s0_ref, s1_ref, inv_ref):
  inv_ref[...] = 1.0 / (s0_ref[...] + s1_ref[...] + 1e-16)


def _tc_inv(ssum):
  return pl.pallas_call(
      _inv_body,
      grid=(NP // _BLK,),
      in_specs=[
          pl.BlockSpec((_BLK, 16), lambda i: (i, 0)),
          pl.BlockSpec((_BLK, 16), lambda i: (i, 0)),
      ],
      out_specs=pl.BlockSpec((_BLK, 16), lambda i: (i, 0)),
      out_shape=jax.ShapeDtypeStruct((NP, 16), _f32),
  )(ssum[0], ssum[1])


def _mid_body(m0_ref, m1_ref, b_ref, w_ref, ms_ref, md_ref,
              x1_ref, xl_ref, tas_ref, tad_ref):
  z = m0_ref[...] + m1_ref[...] + b_ref[...]
  x1 = jnp.where(z > 0, z, jnp.exp(jnp.minimum(z, 0.0)) - 1.0)
  x1_ref[...] = x1
  xl = jnp.dot(x1, w_ref[...], preferred_element_type=_f32)
  xl_ref[...] = xl
  tas_ref[...] = jnp.dot(xl, ms_ref[...], preferred_element_type=_f32)
  tad_ref[...] = jnp.dot(xl, md_ref[...], preferred_element_type=_f32)


def _tc_mid(msg, b1, w2, ms16, md16):
  d = HEADS * OUT
  return pl.pallas_call(
      _mid_body,
      grid=(NP // _BLK,),
      in_specs=[
          pl.BlockSpec((_BLK, d), lambda i: (i, 0)),
          pl.BlockSpec((_BLK, d), lambda i: (i, 0)),
          pl.BlockSpec((1, d), lambda i: (0, 0)),
          pl.BlockSpec((d, d), lambda i: (0, 0)),
          pl.BlockSpec((d, 16), lambda i: (0, 0)),
          pl.BlockSpec((d, 16), lambda i: (0, 0)),
      ],
      out_specs=[
          pl.BlockSpec((_BLK, d), lambda i: (i, 0)),
          pl.BlockSpec((_BLK, d), lambda i: (i, 0)),
          pl.BlockSpec((_BLK, 16), lambda i: (i, 0)),
          pl.BlockSpec((_BLK, 16), lambda i: (i, 0)),
      ],
      out_shape=[
          jax.ShapeDtypeStruct((NP, d), _f32),
          jax.ShapeDtypeStruct((NP, d), _f32),
          jax.ShapeDtypeStruct((NP, 16), _f32),
          jax.ShapeDtypeStruct((NP, 16), _f32),
      ],
  )(msg[0], msg[1], b1.reshape(1, d), w2, ms16, md16)


def _tail_body(x1_ref, m0_ref, m1_ref, b2_ref,
               wf_ref, uf_ref, bf_ref, wb_ref, ub_ref, bb_ref,
               watt_ref, batt_ref, wout_ref, out_ref):
  x1 = x1_ref[...]
  x2 = m0_ref[...] + m1_ref[...] + b2_ref[...]

  wf = wf_ref[...]
  uf = uf_ref[...]
  bf = bf_ref[...]
  wb = wb_ref[...]
  ub = ub_ref[...]
  bb = bb_ref[...]

  def cell(xt, h, c, w, u, b, first):
    g = jnp.dot(xt, w, preferred_element_type=_f32) + b
    if not first:
      g = g + jnp.dot(h, u, preferred_element_type=_f32)
    i = jax.nn.sigmoid(g[:, 0:HID])
    f = jax.nn.sigmoid(g[:, HID:2 * HID])
    gg = jnp.tanh(g[:, 2 * HID:3 * HID])
    o = jax.nn.sigmoid(g[:, 3 * HID:4 * HID])
    c2 = f * c + i * gg
    return o * jnp.tanh(c2), c2

  zero = jnp.zeros_like(x1)
  hf0, cf0 = cell(x1, zero, zero, wf, uf, bf, True)
  hf1, _ = cell(x2, hf0, cf0, wf, uf, bf, False)
  hb1, cb1 = cell(x2, zero, zero, wb, ub, bb, True)
  hb0, _ = cell(x1, hb1, cb1, wb, ub, bb, False)

  watt = watt_ref[...]
  batt = batt_ref[0, 0]
  a0 = jnp.sum(hf0 * watt[:, 0:HID], axis=1, keepdims=True) + \
       jnp.sum(hb0 * watt[:, HID:2 * HID], axis=1, keepdims=True) + batt
  a1 = jnp.sum(hf1 * watt[:, 0:HID], axis=1, keepdims=True) + \
       jnp.sum(hb1 * watt[:, HID:2 * HID], axis=1, keepdims=True) + batt
  m = jnp.maximum(a0, a1)
  e0 = jnp.exp(a0 - m)
  e1 = jnp.exp(a1 - m)
  zs = e0 + e1
  emb = (e0 / zs) * x1 + (e1 / zs) * x2

  logits = jnp.dot(emb, wout_ref[...], preferred_element_type=_f32)
  lm = jnp.max(logits, axis=1, keepdims=True)
  ls = jnp.log(jnp.sum(jnp.exp(logits - lm), axis=1, keepdims=True))
  out_ref[...] = logits - lm - ls


def _tc_tail(x1, msg2, b2, wf, uf, bf, wb, ub, bb, watt, batt, wout):
  d = HEADS * OUT
  return pl.pallas_call(
      _tail_body,
      grid=(NP // _BLK,),
      in_specs=[
          pl.BlockSpec((_BLK, d), lambda i: (i, 0)),
          pl.BlockSpec((_BLK, d), lambda i: (i, 0)),
          pl.BlockSpec((_BLK, d), lambda i: (i, 0)),
          pl.BlockSpec((1, d), lambda i: (0, 0)),
          pl.BlockSpec((HID, 4 * HID), lambda i: (0, 0)),
          pl.BlockSpec((HID, 4 * HID), lambda i: (0, 0)),
          pl.BlockSpec((1, 4 * HID), lambda i: (0, 0)),
          pl.BlockSpec((HID, 4 * HID), lambda i: (0, 0)),
          pl.BlockSpec((HID, 4 * HID), lambda i: (0, 0)),
          pl.BlockSpec((1, 4 * HID), lambda i: (0, 0)),
          pl.BlockSpec((1, 2 * HID), lambda i: (0, 0)),
          pl.BlockSpec((1, 1), lambda i: (0, 0), memory_space=pltpu.SMEM),
          pl.BlockSpec((HID, NUM_CLASSES), lambda i: (0, 0)),
      ],
      out_specs=pl.BlockSpec((_BLK, NUM_CLASSES), lambda i: (i, 0)),
      out_shape=jax.ShapeDtypeStruct((NP, NUM_CLASSES), _f32),
  )(x1, msg2[0], msg2[1], b2.reshape(1, d),
    wf, uf, bf.reshape(1, 4 * HID), wb, ub, bb.reshape(1, 4 * HID),
    watt.reshape(1, 2 * HID), batt.reshape(1, 1), wout)


def _att_mat(a):
  m = jnp.zeros((HEADS * OUT, HEADS), _f32)
  m = m.at[jnp.arange(HEADS * OUT), jnp.arange(HEADS * OUT) // OUT].set(
      a.reshape(-1))
  return jnp.concatenate([m, m], axis=1)


def kernel(x, edge_index, W1, a_src1, a_dst1, b1, W2, a_src2, a_dst2, b2,
           W_ih_f, W_hh_f, b_ih_f, b_hh_f, W_ih_b, W_hh_b, b_ih_b, b_hh_b,
           W_att, b_att, W_out):
  loop = jnp.arange(N, dtype=_i32)
  padv = jnp.full((EP - ETOT,), N, _i32)
  src = jnp.concatenate([edge_index[0].astype(_i32), loop, padv])
  dst = jnp.concatenate([edge_index[1].astype(_i32), loop, padv])
  xp = jnp.concatenate([x, jnp.zeros((NP - N, F_IN), _f32)], axis=0)

  ms1, md1 = _att_mat(a_src1), _att_mat(a_dst1)
  ms2, md2 = _att_mat(a_src2), _att_mat(a_dst2)

  xl1, tas1, tad1 = _tc_proj(xp, W1, ms1, md1)
  p1, ssum1 = _sc_pass1(src, dst, tas1, tad1)
  inv1 = _tc_inv(ssum1)
  msg1 = _sc_pass2(src, dst, p1, inv1, xl1)

  x1, xl2, tas2, tad2 = _tc_mid(msg1, b1, W2, ms2, md2)
  p2, ssum2 = _sc_pass1(src, dst, tas2, tad2)
  inv2 = _tc_inv(ssum2)
  msg2 = _sc_pass2(src, dst, p2, inv2, xl2)

  out = _tc_tail(x1, msg2, b2,
                 W_ih_f.T, W_hh_f.T, b_ih_f + b_hh_f,
                 W_ih_b.T, W_hh_b.T, b_ih_b + b_hh_b,
                 W_att, b_att, W_out)
  return out[:N]


# speedup vs baseline: 58.5338x; 1.3370x over previous
import jax
import jax.numpy as jnp
from jax import lax
from jax.experimental import pallas as pl
from jax.experimental.pallas import tpu as pltpu
from jax.experimental.pallas import tpu_sc as plsc

N = 10000
E = 320000
HEADS = 8
OUT = 8
HID = 64
NUM_CLASSES = 40
F_IN = 128

NP = 10240
C = 128
ITERS = 81
NW = 32
EP = NW * ITERS * C
ETOT = E + N
ROWS_PER_SUB = NP // 16

_f32 = jnp.float32
_i32 = jnp.int32


def _mesh():
  return plsc.VectorSubcoreMesh(
      core_axis_name="c", subcore_axis_name="s", num_cores=2, num_subcores=16)


def _sc_layer_body(src_hbm, dst_hbm, tas_hbm, tad_hbm, xl_hbm,
                   ssum_out, msg_out,
                   sidx2, didx2, a_buf, b_buf, pp_buf, pd_buf, x_buf, m_buf,
                   sacc, macc):
  c = lax.axis_index("c")
  s = lax.axis_index("s")
  wid = s * 2 + c
  lane = lax.iota(_i32, 16)
  lo = lane < 8
  vz = jnp.zeros((16,), _f32)

  pltpu.sync_copy(src_hbm.at[wid], sidx2)
  pltpu.sync_copy(dst_hbm.at[wid], didx2)

  for r in range(C):
    pd_buf[r] = vz
    for v in range(4):
      m_buf[r, pl.ds(16 * v, 16)] = vz
  row0 = s * ROWS_PER_SUB
  for k in range(ROWS_PER_SUB // C):
    pltpu.sync_copy(pd_buf, sacc.at[pl.ds(row0 + k * C, C)])
    pltpu.sync_copy(m_buf, macc.at[pl.ds(row0 + k * C, C)])
  plsc.subcore_barrier()

  idx_lo = jnp.where(lo, lane, lane - 8)
  idx_hi = idx_lo + 8

  def chunk(it, _):
    sidx = sidx2.at[it]
    didx = didx2.at[it]
    pltpu.sync_copy(tas_hbm.at[sidx], a_buf)
    pltpu.sync_copy(tad_hbm.at[didx], b_buf)
    pltpu.sync_copy(xl_hbm.at[sidx], x_buf)
    for j in range(C // 2):
      a0 = a_buf[2 * j]
      a1 = a_buf[2 * j + 1]
      b0 = b_buf[2 * j]
      b1 = b_buf[2 * j + 1]
      al = jnp.where(lo, a0, a1) + jnp.where(lo, b0, b1)
      al = jnp.maximum(al, 0.2 * al)
      pp_buf[j] = jnp.exp(al)
    for j in range(C // 2):
      row = jnp.full((16,), j, _i32)
      pd_buf[2 * j] = plsc.load_gather(pp_buf, [row, idx_lo])
      pd_buf[2 * j + 1] = plsc.load_gather(pp_buf, [row, idx_hi])
      for w in range(8):
        h0 = 2 * (w % 4)
        colv = jnp.where(lo, jnp.full((16,), 8 * (w // 4) + h0, _i32),
                         jnp.full((16,), 8 * (w // 4) + h0 + 1, _i32))
        pv = plsc.load_gather(pp_buf, [row, colv])
        e = 2 * j + (w // 4)
        v = w % 4
        m_buf[e, pl.ds(16 * v, 16)] = pv * x_buf[e, pl.ds(16 * v, 16)]
    pltpu.sync_copy(pd_buf, sacc.at[didx], add=True)
    pltpu.sync_copy(m_buf, macc.at[didx], add=True)
    return _

  lax.fori_loop(0, ITERS, chunk, 0)

  plsc.subcore_barrier()
  pltpu.sync_copy(sacc.at[pl.ds(row0, ROWS_PER_SUB)],
                  ssum_out.at[c, pl.ds(row0, ROWS_PER_SUB)])
  pltpu.sync_copy(macc.at[pl.ds(row0, ROWS_PER_SUB)],
                  msg_out.at[c, pl.ds(row0, ROWS_PER_SUB)])


def _sc_layer(src3, dst3, tas, tad, xl):
  f = pl.kernel(
      _sc_layer_body,
      out_type=[
          jax.ShapeDtypeStruct((2, NP, 16), _f32),
          jax.ShapeDtypeStruct((2, NP, HEADS * OUT), _f32),
      ],
      mesh=_mesh(),
      compiler_params=pltpu.CompilerParams(
          needs_layout_passes=False, use_tc_tiling_on_sc=False),
      scratch_types=[
          pltpu.VMEM((ITERS, C), _i32),
          pltpu.VMEM((ITERS, C), _i32),
          pltpu.VMEM((C, 16), _f32),
          pltpu.VMEM((C, 16), _f32),
          pltpu.VMEM((C // 2, 16), _f32),
          pltpu.VMEM((C, 16), _f32),
          pltpu.VMEM((C, HEADS * OUT), _f32),
          pltpu.VMEM((C, HEADS * OUT), _f32),
          pltpu.VMEM_SHARED((NP, 16), _f32),
          pltpu.VMEM_SHARED((NP, HEADS * OUT), _f32),
      ],
  )
  return f(src3, dst3, tas, tad, xl)


_BLK = 1024


def _proj_body(x_ref, w_ref, ms_ref, md_ref, xl_ref, tas_ref, tad_ref):
  xl = jnp.dot(x_ref[...], w_ref[...], preferred_element_type=_f32)
  xl_ref[...] = xl
  tas_ref[...] = jnp.dot(xl, ms_ref[...], preferred_element_type=_f32)
  tad_ref[...] = jnp.dot(xl, md_ref[...], preferred_element_type=_f32)


def _tc_proj(x, w, ms16, md16):
  fin = x.shape[1]
  return pl.pallas_call(
      _proj_body,
      grid=(NP // _BLK,),
      in_specs=[
          pl.BlockSpec((_BLK, fin), lambda i: (i, 0)),
          pl.BlockSpec((fin, HEADS * OUT), lambda i: (0, 0)),
          pl.BlockSpec((HEADS * OUT, 16), lambda i: (0, 0)),
          pl.BlockSpec((HEADS * OUT, 16), lambda i: (0, 0)),
      ],
      out_specs=[
          pl.BlockSpec((_BLK, HEADS * OUT), lambda i: (i, 0)),
          pl.BlockSpec((_BLK, 16), lambda i: (i, 0)),
          pl.BlockSpec((_BLK, 16), lambda i: (i, 0)),
      ],
      out_shape=[
          jax.ShapeDtypeStruct((NP, HEADS * OUT), _f32),
          jax.ShapeDtypeStruct((NP, 16), _f32),
          jax.ShapeDtypeStruct((NP, 16), _f32),
      ],
  )(x, w, ms16, md16)


def _norm(s0, s1, m0, m1, rep_ref):
  inv8 = 1.0 / (s0[:, 0:HEADS] + s1[:, 0:HEADS] + 1e-16)
  inv64 = jnp.dot(inv8, rep_ref, preferred_element_type=_f32)
  return (m0 + m1) * inv64


def _mid_body(s0_ref, s1_ref, m0_ref, m1_ref, rep_ref, b_ref,
              w_ref, ms_ref, md_ref,
              x1_ref, xl_ref, tas_ref, tad_ref):
  z = _norm(s0_ref[...], s1_ref[...], m0_ref[...], m1_ref[...],
            rep_ref[...]) + b_ref[...]
  x1 = jnp.where(z > 0, z, jnp.exp(jnp.minimum(z, 0.0)) - 1.0)
  x1_ref[...] = x1
  xl = jnp.dot(x1, w_ref[...], preferred_element_type=_f32)
  xl_ref[...] = xl
  tas_ref[...] = jnp.dot(xl, ms_ref[...], preferred_element_type=_f32)
  tad_ref[...] = jnp.dot(xl, md_ref[...], preferred_element_type=_f32)


def _tc_mid(ssum, msg, rep8, b1, w2, ms16, md16):
  d = HEADS * OUT
  return pl.pallas_call(
      _mid_body,
      grid=(NP // _BLK,),
      in_specs=[
          pl.BlockSpec((_BLK, 16), lambda i: (i, 0)),
          pl.BlockSpec((_BLK, 16), lambda i: (i, 0)),
          pl.BlockSpec((_BLK, d), lambda i: (i, 0)),
          pl.BlockSpec((_BLK, d), lambda i: (i, 0)),
          pl.BlockSpec((HEADS, d), lambda i: (0, 0)),
          pl.BlockSpec((1, d), lambda i: (0, 0)),
          pl.BlockSpec((d, d), lambda i: (0, 0)),
          pl.BlockSpec((d, 16), lambda i: (0, 0)),
          pl.BlockSpec((d, 16), lambda i: (0, 0)),
      ],
      out_specs=[
          pl.BlockSpec((_BLK, d), lambda i: (i, 0)),
          pl.BlockSpec((_BLK, d), lambda i: (i, 0)),
          pl.BlockSpec((_BLK, 16), lambda i: (i, 0)),
          pl.BlockSpec((_BLK, 16), lambda i: (i, 0)),
      ],
      out_shape=[
          jax.ShapeDtypeStruct((NP, d), _f32),
          jax.ShapeDtypeStruct((NP, d), _f32),
          jax.ShapeDtypeStruct((NP, 16), _f32),
          jax.ShapeDtypeStruct((NP, 16), _f32),
      ],
  )(ssum[0], ssum[1], msg[0], msg[1], rep8, b1.reshape(1, d), w2, ms16, md16)


def _tail_body(x1_ref, s0_ref, s1_ref, m0_ref, m1_ref, rep_ref, b2_ref,
               wf_ref, uf_ref, bf_ref, wb_ref, ub_ref, bb_ref,
               watt_ref, batt_ref, wout_ref, out_ref):
  x1 = x1_ref[...]
  x2 = _norm(s0_ref[...], s1_ref[...], m0_ref[...], m1_ref[...],
             rep_ref[...]) + b2_ref[...]

  wf = wf_ref[...]
  uf = uf_ref[...]
  bf = bf_ref[...]
  wb = wb_ref[...]
  ub = ub_ref[...]
  bb = bb_ref[...]

  def cell(xt, h, c, w, u, b, first):
    g = jnp.dot(xt, w, preferred_element_type=_f32) + b
    if not first:
      g = g + jnp.dot(h, u, preferred_element_type=_f32)
    i = jax.nn.sigmoid(g[:, 0:HID])
    f = jax.nn.sigmoid(g[:, HID:2 * HID])
    gg = jnp.tanh(g[:, 2 * HID:3 * HID])
    o = jax.nn.sigmoid(g[:, 3 * HID:4 * HID])
    c2 = f * c + i * gg
    return o * jnp.tanh(c2), c2

  zero = jnp.zeros_like(x1)
  hf0, cf0 = cell(x1, zero, zero, wf, uf, bf, True)
  hf1, _ = cell(x2, hf0, cf0, wf, uf, bf, False)
  hb1, cb1 = cell(x2, zero, zero, wb, ub, bb, True)
  hb0, _ = cell(x1, hb1, cb1, wb, ub, bb, False)

  watt = watt_ref[...]
  batt = batt_ref[0, 0]
  a0 = jnp.sum(hf0 * watt[:, 0:HID], axis=1, keepdims=True) + \
       jnp.sum(hb0 * watt[:, HID:2 * HID], axis=1, keepdims=True) + batt
  a1 = jnp.sum(hf1 * watt[:, 0:HID], axis=1, keepdims=True) + \
       jnp.sum(hb1 * watt[:, HID:2 * HID], axis=1, keepdims=True) + batt
  m = jnp.maximum(a0, a1)
  e0 = jnp.exp(a0 - m)
  e1 = jnp.exp(a1 - m)
  zs = e0 + e1
  emb = (e0 / zs) * x1 + (e1 / zs) * x2

  logits = jnp.dot(emb, wout_ref[...], preferred_element_type=_f32)
  lm = jnp.max(logits, axis=1, keepdims=True)
  ls = jnp.log(jnp.sum(jnp.exp(logits - lm), axis=1, keepdims=True))
  out_ref[...] = logits - lm - ls


def _tc_tail(x1, ssum2, msg2, rep8, b2, wf, uf, bf, wb, ub, bb,
             watt, batt, wout):
  d = HEADS * OUT
  return pl.pallas_call(
      _tail_body,
      grid=(NP // _BLK,),
      in_specs=[
          pl.BlockSpec((_BLK, d), lambda i: (i, 0)),
          pl.BlockSpec((_BLK, 16), lambda i: (i, 0)),
          pl.BlockSpec((_BLK, 16), lambda i: (i, 0)),
          pl.BlockSpec((_BLK, d), lambda i: (i, 0)),
          pl.BlockSpec((_BLK, d), lambda i: (i, 0)),
          pl.BlockSpec((HEADS, d), lambda i: (0, 0)),
          pl.BlockSpec((1, d), lambda i: (0, 0)),
          pl.BlockSpec((HID, 4 * HID), lambda i: (0, 0)),
          pl.BlockSpec((HID, 4 * HID), lambda i: (0, 0)),
          pl.BlockSpec((1, 4 * HID), lambda i: (0, 0)),
          pl.BlockSpec((HID, 4 * HID), lambda i: (0, 0)),
          pl.BlockSpec((HID, 4 * HID), lambda i: (0, 0)),
          pl.BlockSpec((1, 4 * HID), lambda i: (0, 0)),
          pl.BlockSpec((1, 2 * HID), lambda i: (0, 0)),
          pl.BlockSpec((1, 1), lambda i: (0, 0), memory_space=pltpu.SMEM),
          pl.BlockSpec((HID, NUM_CLASSES), lambda i: (0, 0)),
      ],
      out_specs=pl.BlockSpec((_BLK, NUM_CLASSES), lambda i: (i, 0)),
      out_shape=jax.ShapeDtypeStruct((NP, NUM_CLASSES), _f32),
  )(x1, ssum2[0], ssum2[1], msg2[0], msg2[1], rep8, b2.reshape(1, d),
    wf, uf, bf.reshape(1, 4 * HID), wb, ub, bb.reshape(1, 4 * HID),
    watt.reshape(1, 2 * HID), batt.reshape(1, 1), wout)


def _att_mat(a):
  m = jnp.zeros((HEADS * OUT, HEADS), _f32)
  m = m.at[jnp.arange(HEADS * OUT), jnp.arange(HEADS * OUT) // OUT].set(
      a.reshape(-1))
  return jnp.concatenate([m, m], axis=1)


def kernel(x, edge_index, W1, a_src1, a_dst1, b1, W2, a_src2, a_dst2, b2,
           W_ih_f, W_hh_f, b_ih_f, b_hh_f, W_ih_b, W_hh_b, b_ih_b, b_hh_b,
           W_att, b_att, W_out):
  loop = jnp.arange(N, dtype=_i32)
  padv = jnp.full((EP - ETOT,), N, _i32)
  src3 = jnp.concatenate([edge_index[0].astype(_i32), loop, padv]
                         ).reshape(NW, ITERS, C)
  dst3 = jnp.concatenate([edge_index[1].astype(_i32), loop, padv]
                         ).reshape(NW, ITERS, C)
  xp = jnp.concatenate([x, jnp.zeros((NP - N, F_IN), _f32)], axis=0)

  ms1, md1 = _att_mat(a_src1), _att_mat(a_dst1)
  ms2, md2 = _att_mat(a_src2), _att_mat(a_dst2)
  rep8 = jnp.kron(jnp.eye(HEADS, dtype=_f32), jnp.ones((1, OUT), _f32))

  xl1, tas1, tad1 = _tc_proj(xp, W1, ms1, md1)
  ssum1, msg1 = _sc_layer(src3, dst3, tas1, tad1, xl1)

  x1, xl2, tas2, tad2 = _tc_mid(ssum1, msg1, rep8, b1, W2, ms2, md2)
  ssum2, msg2 = _sc_layer(src3, dst3, tas2, tad2, xl2)

  out = _tc_tail(x1, ssum2, msg2, rep8, b2,
                 W_ih_f.T, W_hh_f.T, b_ih_f + b_hh_f,
                 W_ih_b.T, W_hh_b.T, b_ih_b + b_hh_b,
                 W_att, b_att, W_out)
  return out[:N]


# speedup vs baseline: 66.7836x; 1.1409x over previous
import jax
import jax.numpy as jnp
from jax import lax
from jax.experimental import pallas as pl
from jax.experimental.pallas import tpu as pltpu
from jax.experimental.pallas import tpu_sc as plsc

N = 10000
E = 320000
HEADS = 8
OUT = 8
HID = 64
NUM_CLASSES = 40
F_IN = 128

NP = 10240
C = 128
ITERS = 82
NW = 32
EP = NW * ITERS * C
ETOT = E + N
ROWS_PER_SUB = NP // 16

_f32 = jnp.float32
_i32 = jnp.int32


def _mesh():
  return plsc.VectorSubcoreMesh(
      core_axis_name="c", subcore_axis_name="s", num_cores=2, num_subcores=16)


def _sc_layer_body(src_hbm, dst_hbm, tas_hbm, tad_hbm, xl_hbm,
                   ssum_out, msg_out,
                   sidx2, didx2, a_buf, b_buf, pp_buf, pd_buf, x_buf, m_buf,
                   sems, sacc, macc):
  c = lax.axis_index("c")
  s = lax.axis_index("s")
  wid = s * 2 + c
  lane = lax.iota(_i32, 16)
  lo = lane < 8
  vz = jnp.zeros((16,), _f32)

  pltpu.sync_copy(src_hbm.at[wid], sidx2)
  pltpu.sync_copy(dst_hbm.at[wid], didx2)

  for r in range(C):
    pd_buf[0, r] = vz
    for v in range(4):
      m_buf[0, r, pl.ds(16 * v, 16)] = vz
  row0 = s * ROWS_PER_SUB
  for k in range(ROWS_PER_SUB // C):
    pltpu.sync_copy(pd_buf.at[0], sacc.at[pl.ds(row0 + k * C, C)])
    pltpu.sync_copy(m_buf.at[0], macc.at[pl.ds(row0 + k * C, C)])
  plsc.subcore_barrier()

  idx_lo = jnp.where(lo, lane, lane - 8)
  idx_hi = idx_lo + 8

  def issue_in(it, slot):
    pltpu.async_copy(tas_hbm.at[sidx2.at[it]], a_buf.at[slot],
                     sems.at[slot, 0])
    pltpu.async_copy(tad_hbm.at[didx2.at[it]], b_buf.at[slot],
                     sems.at[slot, 1])
    pltpu.async_copy(xl_hbm.at[sidx2.at[it]], x_buf.at[slot],
                     sems.at[slot, 2])

  def wait_in(it, slot):
    pltpu.make_async_copy(tas_hbm.at[sidx2.at[it]], a_buf.at[slot],
                          sems.at[slot, 0]).wait()
    pltpu.make_async_copy(tad_hbm.at[didx2.at[it]], b_buf.at[slot],
                          sems.at[slot, 1]).wait()
    pltpu.make_async_copy(xl_hbm.at[sidx2.at[it]], x_buf.at[slot],
                          sems.at[slot, 2]).wait()

  def issue_scatter(it, slot):
    pltpu.async_copy(pd_buf.at[slot], sacc.at[didx2.at[it]],
                     sems.at[slot, 3], add=True)
    pltpu.async_copy(m_buf.at[slot], macc.at[didx2.at[it]],
                     sems.at[slot, 4], add=True)

  def wait_scatter(it, slot):
    pltpu.make_async_copy(pd_buf.at[slot], sacc.at[didx2.at[it]],
                          sems.at[slot, 3]).wait()
    pltpu.make_async_copy(m_buf.at[slot], macc.at[didx2.at[it]],
                          sems.at[slot, 4]).wait()

  def compute(slot):
    for j in range(C // 2):
      a0 = a_buf[slot, 2 * j]
      a1 = a_buf[slot, 2 * j + 1]
      b0 = b_buf[slot, 2 * j]
      b1 = b_buf[slot, 2 * j + 1]
      al = jnp.where(lo, a0, a1) + jnp.where(lo, b0, b1)
      al = jnp.maximum(al, 0.2 * al)
      pp_buf[j] = jnp.exp(al)
    for j in range(C // 2):
      row = jnp.full((16,), j, _i32)
      pd_buf[slot, 2 * j] = plsc.load_gather(pp_buf, [row, idx_lo])
      pd_buf[slot, 2 * j + 1] = plsc.load_gather(pp_buf, [row, idx_hi])
      for w in range(8):
        h0 = 2 * (w % 4)
        colv = jnp.where(lo, jnp.full((16,), 8 * (w // 4) + h0, _i32),
                         jnp.full((16,), 8 * (w // 4) + h0 + 1, _i32))
        pv = plsc.load_gather(pp_buf, [row, colv])
        e = 2 * j + (w // 4)
        v = w % 4
        m_buf[slot, e, pl.ds(16 * v, 16)] = pv * x_buf[slot, e,
                                                       pl.ds(16 * v, 16)]

  issue_in(0, 0)

  def pair(t, _):
    it0 = t * 2
    it1 = it0 + 1
    issue_in(it1, 1)
    wait_in(it0, 0)

    @pl.when(t > 0)
    def _w0():
      wait_scatter(it0 - 2, 0)

    compute(0)
    issue_scatter(it0, 0)

    @pl.when(t < ITERS // 2 - 1)
    def _p0():
      issue_in(it0 + 2, 0)

    wait_in(it1, 1)

    @pl.when(t > 0)
    def _w1():
      wait_scatter(it1 - 2, 1)

    compute(1)
    issue_scatter(it1, 1)
    return _

  lax.fori_loop(0, ITERS // 2, pair, 0)
  wait_scatter(ITERS - 2, 0)
  wait_scatter(ITERS - 1, 1)

  plsc.subcore_barrier()
  pltpu.sync_copy(sacc.at[pl.ds(row0, ROWS_PER_SUB)],
                  ssum_out.at[c, pl.ds(row0, ROWS_PER_SUB)])
  pltpu.sync_copy(macc.at[pl.ds(row0, ROWS_PER_SUB)],
                  msg_out.at[c, pl.ds(row0, ROWS_PER_SUB)])


def _sc_layer(src3, dst3, tas, tad, xl):
  f = pl.kernel(
      _sc_layer_body,
      out_type=[
          jax.ShapeDtypeStruct((2, NP, 16), _f32),
          jax.ShapeDtypeStruct((2, NP, HEADS * OUT), _f32),
      ],
      mesh=_mesh(),
      compiler_params=pltpu.CompilerParams(
          needs_layout_passes=False, use_tc_tiling_on_sc=False),
      scratch_types=[
          pltpu.VMEM((ITERS, C), _i32),
          pltpu.VMEM((ITERS, C), _i32),
          pltpu.VMEM((2, C, 16), _f32),
          pltpu.VMEM((2, C, 16), _f32),
          pltpu.VMEM((C // 2, 16), _f32),
          pltpu.VMEM((2, C, 16), _f32),
          pltpu.VMEM((2, C, HEADS * OUT), _f32),
          pltpu.VMEM((2, C, HEADS * OUT), _f32),
          pltpu.SemaphoreType.DMA((2, 5)),
          pltpu.VMEM_SHARED((NP, 16), _f32),
          pltpu.VMEM_SHARED((NP, HEADS * OUT), _f32),
      ],
  )
  return f(src3, dst3, tas, tad, xl)


_BLK = 1024


def _proj_body(x_ref, w_ref, ms_ref, md_ref, xl_ref, tas_ref, tad_ref):
  xl = jnp.dot(x_ref[...], w_ref[...], preferred_element_type=_f32)
  xl_ref[...] = xl
  tas_ref[...] = jnp.dot(xl, ms_ref[...], preferred_element_type=_f32)
  tad_ref[...] = jnp.dot(xl, md_ref[...], preferred_element_type=_f32)


def _tc_proj(x, w, ms16, md16):
  fin = x.shape[1]
  return pl.pallas_call(
      _proj_body,
      grid=(NP // _BLK,),
      in_specs=[
          pl.BlockSpec((_BLK, fin), lambda i: (i, 0)),
          pl.BlockSpec((fin, HEADS * OUT), lambda i: (0, 0)),
          pl.BlockSpec((HEADS * OUT, 16), lambda i: (0, 0)),
          pl.BlockSpec((HEADS * OUT, 16), lambda i: (0, 0)),
      ],
      out_specs=[
          pl.BlockSpec((_BLK, HEADS * OUT), lambda i: (i, 0)),
          pl.BlockSpec((_BLK, 16), lambda i: (i, 0)),
          pl.BlockSpec((_BLK, 16), lambda i: (i, 0)),
      ],
      out_shape=[
          jax.ShapeDtypeStruct((NP, HEADS * OUT), _f32),
          jax.ShapeDtypeStruct((NP, 16), _f32),
          jax.ShapeDtypeStruct((NP, 16), _f32),
      ],
  )(x, w, ms16, md16)


def _norm(s0, s1, m0, m1, rep_ref):
  inv8 = 1.0 / (s0[:, 0:HEADS] + s1[:, 0:HEADS] + 1e-16)
  inv64 = jnp.dot(inv8, rep_ref, preferred_element_type=_f32)
  return (m0 + m1) * inv64


def _mid_body(s0_ref, s1_ref, m0_ref, m1_ref, rep_ref, b_ref,
              w_ref, ms_ref, md_ref,
              x1_ref, xl_ref, tas_ref, tad_ref):
  z = _norm(s0_ref[...], s1_ref[...], m0_ref[...], m1_ref[...],
            rep_ref[...]) + b_ref[...]
  x1 = jnp.where(z > 0, z, jnp.exp(jnp.minimum(z, 0.0)) - 1.0)
  x1_ref[...] = x1
  xl = jnp.dot(x1, w_ref[...], preferred_element_type=_f32)
  xl_ref[...] = xl
  tas_ref[...] = jnp.dot(xl, ms_ref[...], preferred_element_type=_f32)
  tad_ref[...] = jnp.dot(xl, md_ref[...], preferred_element_type=_f32)


def _tc_mid(ssum, msg, rep8, b1, w2, ms16, md16):
  d = HEADS * OUT
  return pl.pallas_call(
      _mid_body,
      grid=(NP // _BLK,),
      in_specs=[
          pl.BlockSpec((_BLK, 16), lambda i: (i, 0)),
          pl.BlockSpec((_BLK, 16), lambda i: (i, 0)),
          pl.BlockSpec((_BLK, d), lambda i: (i, 0)),
          pl.BlockSpec((_BLK, d), lambda i: (i, 0)),
          pl.BlockSpec((HEADS, d), lambda i: (0, 0)),
          pl.BlockSpec((1, d), lambda i: (0, 0)),
          pl.BlockSpec((d, d), lambda i: (0, 0)),
          pl.BlockSpec((d, 16), lambda i: (0, 0)),
          pl.BlockSpec((d, 16), lambda i: (0, 0)),
      ],
      out_specs=[
          pl.BlockSpec((_BLK, d), lambda i: (i, 0)),
          pl.BlockSpec((_BLK, d), lambda i: (i, 0)),
          pl.BlockSpec((_BLK, 16), lambda i: (i, 0)),
          pl.BlockSpec((_BLK, 16), lambda i: (i, 0)),
      ],
      out_shape=[
          jax.ShapeDtypeStruct((NP, d), _f32),
          jax.ShapeDtypeStruct((NP, d), _f32),
          jax.ShapeDtypeStruct((NP, 16), _f32),
          jax.ShapeDtypeStruct((NP, 16), _f32),
      ],
  )(ssum[0], ssum[1], msg[0], msg[1], rep8, b1.reshape(1, d), w2, ms16, md16)


def _tail_body(x1_ref, s0_ref, s1_ref, m0_ref, m1_ref, rep_ref, b2_ref,
               wf_ref, uf_ref, bf_ref, wb_ref, ub_ref, bb_ref,
               watt_ref, batt_ref, wout_ref, out_ref):
  x1 = x1_ref[...]
  x2 = _norm(s0_ref[...], s1_ref[...], m0_ref[...], m1_ref[...],
             rep_ref[...]) + b2_ref[...]

  wf = wf_ref[...]
  uf = uf_ref[...]
  bf = bf_ref[...]
  wb = wb_ref[...]
  ub = ub_ref[...]
  bb = bb_ref[...]

  def cell(xt, h, c, w, u, b, first):
    g = jnp.dot(xt, w, preferred_element_type=_f32) + b
    if not first:
      g = g + jnp.dot(h, u, preferred_element_type=_f32)
    i = jax.nn.sigmoid(g[:, 0:HID])
    f = jax.nn.sigmoid(g[:, HID:2 * HID])
    gg = jnp.tanh(g[:, 2 * HID:3 * HID])
    o = jax.nn.sigmoid(g[:, 3 * HID:4 * HID])
    c2 = f * c + i * gg
    return o * jnp.tanh(c2), c2

  zero = jnp.zeros_like(x1)
  hf0, cf0 = cell(x1, zero, zero, wf, uf, bf, True)
  hf1, _ = cell(x2, hf0, cf0, wf, uf, bf, False)
  hb1, cb1 = cell(x2, zero, zero, wb, ub, bb, True)
  hb0, _ = cell(x1, hb1, cb1, wb, ub, bb, False)

  watt = watt_ref[...]
  batt = batt_ref[0, 0]
  a0 = jnp.sum(hf0 * watt[:, 0:HID], axis=1, keepdims=True) + \
       jnp.sum(hb0 * watt[:, HID:2 * HID], axis=1, keepdims=True) + batt
  a1 = jnp.sum(hf1 * watt[:, 0:HID], axis=1, keepdims=True) + \
       jnp.sum(hb1 * watt[:, HID:2 * HID], axis=1, keepdims=True) + batt
  m = jnp.maximum(a0, a1)
  e0 = jnp.exp(a0 - m)
  e1 = jnp.exp(a1 - m)
  zs = e0 + e1
  emb = (e0 / zs) * x1 + (e1 / zs) * x2

  logits = jnp.dot(emb, wout_ref[...], preferred_element_type=_f32)
  lm = jnp.max(logits, axis=1, keepdims=True)
  ls = jnp.log(jnp.sum(jnp.exp(logits - lm), axis=1, keepdims=True))
  out_ref[...] = logits - lm - ls


def _tc_tail(x1, ssum2, msg2, rep8, b2, wf, uf, bf, wb, ub, bb,
             watt, batt, wout):
  d = HEADS * OUT
  return pl.pallas_call(
      _tail_body,
      grid=(NP // _BLK,),
      in_specs=[
          pl.BlockSpec((_BLK, d), lambda i: (i, 0)),
          pl.BlockSpec((_BLK, 16), lambda i: (i, 0)),
          pl.BlockSpec((_BLK, 16), lambda i: (i, 0)),
          pl.BlockSpec((_BLK, d), lambda i: (i, 0)),
          pl.BlockSpec((_BLK, d), lambda i: (i, 0)),
          pl.BlockSpec((HEADS, d), lambda i: (0, 0)),
          pl.BlockSpec((1, d), lambda i: (0, 0)),
          pl.BlockSpec((HID, 4 * HID), lambda i: (0, 0)),
          pl.BlockSpec((HID, 4 * HID), lambda i: (0, 0)),
          pl.BlockSpec((1, 4 * HID), lambda i: (0, 0)),
          pl.BlockSpec((HID, 4 * HID), lambda i: (0, 0)),
          pl.BlockSpec((HID, 4 * HID), lambda i: (0, 0)),
          pl.BlockSpec((1, 4 * HID), lambda i: (0, 0)),
          pl.BlockSpec((1, 2 * HID), lambda i: (0, 0)),
          pl.BlockSpec((1, 1), lambda i: (0, 0), memory_space=pltpu.SMEM),
          pl.BlockSpec((HID, NUM_CLASSES), lambda i: (0, 0)),
      ],
      out_specs=pl.BlockSpec((_BLK, NUM_CLASSES), lambda i: (i, 0)),
      out_shape=jax.ShapeDtypeStruct((NP, NUM_CLASSES), _f32),
  )(x1, ssum2[0], ssum2[1], msg2[0], msg2[1], rep8, b2.reshape(1, d),
    wf, uf, bf.reshape(1, 4 * HID), wb, ub, bb.reshape(1, 4 * HID),
    watt.reshape(1, 2 * HID), batt.reshape(1, 1), wout)


def _att_mat(a):
  m = jnp.zeros((HEADS * OUT, HEADS), _f32)
  m = m.at[jnp.arange(HEADS * OUT), jnp.arange(HEADS * OUT) // OUT].set(
      a.reshape(-1))
  return jnp.concatenate([m, m], axis=1)


def kernel(x, edge_index, W1, a_src1, a_dst1, b1, W2, a_src2, a_dst2, b2,
           W_ih_f, W_hh_f, b_ih_f, b_hh_f, W_ih_b, W_hh_b, b_ih_b, b_hh_b,
           W_att, b_att, W_out):
  loop = jnp.arange(N, dtype=_i32)
  padv = jnp.full((EP - ETOT,), N, _i32)
  src3 = jnp.concatenate([edge_index[0].astype(_i32), loop, padv]
                         ).reshape(NW, ITERS, C)
  dst3 = jnp.concatenate([edge_index[1].astype(_i32), loop, padv]
                         ).reshape(NW, ITERS, C)
  xp = jnp.concatenate([x, jnp.zeros((NP - N, F_IN), _f32)], axis=0)

  ms1, md1 = _att_mat(a_src1), _att_mat(a_dst1)
  ms2, md2 = _att_mat(a_src2), _att_mat(a_dst2)
  rep8 = jnp.kron(jnp.eye(HEADS, dtype=_f32), jnp.ones((1, OUT), _f32))

  xl1, tas1, tad1 = _tc_proj(xp, W1, ms1, md1)
  ssum1, msg1 = _sc_layer(src3, dst3, tas1, tad1, xl1)

  x1, xl2, tas2, tad2 = _tc_mid(ssum1, msg1, rep8, b1, W2, ms2, md2)
  ssum2, msg2 = _sc_layer(src3, dst3, tas2, tad2, xl2)

  out = _tc_tail(x1, ssum2, msg2, rep8, b2,
                 W_ih_f.T, W_hh_f.T, b_ih_f + b_hh_f,
                 W_ih_b.T, W_hh_b.T, b_ih_b + b_hh_b,
                 W_att, b_att, W_out)
  return out[:N]


# speedup vs baseline: 69.4445x; 1.0398x over previous
import jax
import jax.numpy as jnp
from jax import lax
from jax.experimental import pallas as pl
from jax.experimental.pallas import tpu as pltpu
from jax.experimental.pallas import tpu_sc as plsc

N = 10000
E = 320000
HEADS = 8
OUT = 8
HID = 64
NUM_CLASSES = 40
F_IN = 128

NP = 10240
C = 128
ITERS = 82
NW = 32
EP = NW * ITERS * C
ETOT = E + N
ROWS_PER_SUB = NP // 16
FW = 80

_f32 = jnp.float32
_i32 = jnp.int32


def _mesh():
  return plsc.VectorSubcoreMesh(
      core_axis_name="c", subcore_axis_name="s", num_cores=2, num_subcores=16)


def _sc_layer_body(src_hbm, dst_hbm, xa_hbm, tad_hbm,
                   msg_out,
                   sidx2, didx2, b_buf, pp_buf, x_buf, m_buf,
                   sems, macc):
  c = lax.axis_index("c")
  s = lax.axis_index("s")
  wid = s * 2 + c
  lane = lax.iota(_i32, 16)
  lo = lane < 8
  vz = jnp.zeros((16,), _f32)

  pltpu.sync_copy(src_hbm.at[wid], sidx2)
  pltpu.sync_copy(dst_hbm.at[wid], didx2)

  for r in range(C):
    for v in range(FW // 16):
      m_buf[0, r, pl.ds(16 * v, 16)] = vz
  row0 = s * ROWS_PER_SUB
  for k in range(ROWS_PER_SUB // C):
    pltpu.sync_copy(m_buf.at[0], macc.at[pl.ds(row0 + k * C, C)])
  plsc.subcore_barrier()

  idx_lo = jnp.where(lo, lane, lane - 8)
  idx_hi = idx_lo + 8

  def issue_in(it, slot):
    pltpu.async_copy(xa_hbm.at[sidx2.at[it]], x_buf.at[slot],
                     sems.at[slot, 0])
    pltpu.async_copy(tad_hbm.at[didx2.at[it]], b_buf.at[slot],
                     sems.at[slot, 1])

  def wait_in(it, slot):
    pltpu.make_async_copy(xa_hbm.at[sidx2.at[it]], x_buf.at[slot],
                          sems.at[slot, 0]).wait()
    pltpu.make_async_copy(tad_hbm.at[didx2.at[it]], b_buf.at[slot],
                          sems.at[slot, 1]).wait()

  def issue_scatter(it, slot):
    pltpu.async_copy(m_buf.at[slot], macc.at[didx2.at[it]],
                     sems.at[slot, 2], add=True)

  def wait_scatter(it, slot):
    pltpu.make_async_copy(m_buf.at[slot], macc.at[didx2.at[it]],
                          sems.at[slot, 2]).wait()

  def compute(slot):
    for j in range(C // 2):
      a0 = x_buf[slot, 2 * j, pl.ds(64, 16)]
      a1 = x_buf[slot, 2 * j + 1, pl.ds(64, 16)]
      b0 = b_buf[slot, 2 * j]
      b1 = b_buf[slot, 2 * j + 1]
      al = jnp.where(lo, a0, a1) + jnp.where(lo, b0, b1)
      al = jnp.maximum(al, 0.2 * al)
      pp_buf[j] = jnp.exp(al)
    for j in range(C // 2):
      row = jnp.full((16,), j, _i32)
      m_buf[slot, 2 * j, pl.ds(64, 16)] = plsc.load_gather(
          pp_buf, [row, idx_lo])
      m_buf[slot, 2 * j + 1, pl.ds(64, 16)] = plsc.load_gather(
          pp_buf, [row, idx_hi])
      for w in range(8):
        h0 = 2 * (w % 4)
        colv = jnp.where(lo, jnp.full((16,), 8 * (w // 4) + h0, _i32),
                         jnp.full((16,), 8 * (w // 4) + h0 + 1, _i32))
        pv = plsc.load_gather(pp_buf, [row, colv])
        e = 2 * j + (w // 4)
        v = w % 4
        m_buf[slot, e, pl.ds(16 * v, 16)] = pv * x_buf[slot, e,
                                                       pl.ds(16 * v, 16)]

  issue_in(0, 0)

  def pair(t, _):
    it0 = t * 2
    it1 = it0 + 1
    issue_in(it1, 1)
    wait_in(it0, 0)

    @pl.when(t > 0)
    def _w0():
      wait_scatter(it0 - 2, 0)

    compute(0)
    issue_scatter(it0, 0)

    @pl.when(t < ITERS // 2 - 1)
    def _p0():
      issue_in(it0 + 2, 0)

    wait_in(it1, 1)

    @pl.when(t > 0)
    def _w1():
      wait_scatter(it1 - 2, 1)

    compute(1)
    issue_scatter(it1, 1)
    return _

  lax.fori_loop(0, ITERS // 2, pair, 0)
  wait_scatter(ITERS - 2, 0)
  wait_scatter(ITERS - 1, 1)

  plsc.subcore_barrier()
  pltpu.sync_copy(macc.at[pl.ds(row0, ROWS_PER_SUB)],
                  msg_out.at[c, pl.ds(row0, ROWS_PER_SUB)])


def _sc_layer(src3, dst3, xa, tad):
  f = pl.kernel(
      _sc_layer_body,
      out_type=jax.ShapeDtypeStruct((2, NP, FW), _f32),
      mesh=_mesh(),
      compiler_params=pltpu.CompilerParams(
          needs_layout_passes=False, use_tc_tiling_on_sc=False),
      scratch_types=[
          pltpu.VMEM((ITERS, C), _i32),
          pltpu.VMEM((ITERS, C), _i32),
          pltpu.VMEM((2, C, 16), _f32),
          pltpu.VMEM((C // 2, 16), _f32),
          pltpu.VMEM((2, C, FW), _f32),
          pltpu.VMEM((2, C, FW), _f32),
          pltpu.SemaphoreType.DMA((2, 3)),
          pltpu.VMEM_SHARED((NP, FW), _f32),
      ],
  )
  return f(src3, dst3, xa, tad)


_BLK = 1024


def _proj_body(x_ref, w_ref, ms_ref, md_ref, xa_ref, tad_ref):
  xl = jnp.dot(x_ref[...], w_ref[...], preferred_element_type=_f32)
  xa_ref[:, 0:HEADS * OUT] = xl
  xa_ref[:, HEADS * OUT:FW] = jnp.dot(xl, ms_ref[...],
                                      preferred_element_type=_f32)
  tad_ref[...] = jnp.dot(xl, md_ref[...], preferred_element_type=_f32)


def _tc_proj(x, w, ms16, md16):
  fin = x.shape[1]
  return pl.pallas_call(
      _proj_body,
      grid=(NP // _BLK,),
      in_specs=[
          pl.BlockSpec((_BLK, fin), lambda i: (i, 0)),
          pl.BlockSpec((fin, HEADS * OUT), lambda i: (0, 0)),
          pl.BlockSpec((HEADS * OUT, 16), lambda i: (0, 0)),
          pl.BlockSpec((HEADS * OUT, 16), lambda i: (0, 0)),
      ],
      out_specs=[
          pl.BlockSpec((_BLK, FW), lambda i: (i, 0)),
          pl.BlockSpec((_BLK, 16), lambda i: (i, 0)),
      ],
      out_shape=[
          jax.ShapeDtypeStruct((NP, FW), _f32),
          jax.ShapeDtypeStruct((NP, 16), _f32),
      ],
  )(x, w, ms16, md16)


def _norm(m0, m1, rep_ref):
  m = m0 + m1
  inv8 = 1.0 / (m[:, HEADS * OUT:HEADS * OUT + HEADS] + 1e-16)
  inv64 = jnp.dot(inv8, rep_ref, preferred_element_type=_f32)
  return m[:, 0:HEADS * OUT] * inv64


def _mid_body(m0_ref, m1_ref, rep_ref, b_ref,
              w_ref, ms_ref, md_ref,
              x1_ref, xa_ref, tad_ref):
  z = _norm(m0_ref[...], m1_ref[...], rep_ref[...]) + b_ref[...]
  x1 = jnp.where(z > 0, z, jnp.exp(jnp.minimum(z, 0.0)) - 1.0)
  x1_ref[...] = x1
  xl = jnp.dot(x1, w_ref[...], preferred_element_type=_f32)
  xa_ref[:, 0:HEADS * OUT] = xl
  xa_ref[:, HEADS * OUT:FW] = jnp.dot(xl, ms_ref[...],
                                      preferred_element_type=_f32)
  tad_ref[...] = jnp.dot(xl, md_ref[...], preferred_element_type=_f32)


def _tc_mid(msg, rep8, b1, w2, ms16, md16):
  d = HEADS * OUT
  return pl.pallas_call(
      _mid_body,
      grid=(NP // _BLK,),
      in_specs=[
          pl.BlockSpec((_BLK, FW), lambda i: (i, 0)),
          pl.BlockSpec((_BLK, FW), lambda i: (i, 0)),
          pl.BlockSpec((HEADS, d), lambda i: (0, 0)),
          pl.BlockSpec((1, d), lambda i: (0, 0)),
          pl.BlockSpec((d, d), lambda i: (0, 0)),
          pl.BlockSpec((d, 16), lambda i: (0, 0)),
          pl.BlockSpec((d, 16), lambda i: (0, 0)),
      ],
      out_specs=[
          pl.BlockSpec((_BLK, d), lambda i: (i, 0)),
          pl.BlockSpec((_BLK, FW), lambda i: (i, 0)),
          pl.BlockSpec((_BLK, 16), lambda i: (i, 0)),
      ],
      out_shape=[
          jax.ShapeDtypeStruct((NP, d), _f32),
          jax.ShapeDtypeStruct((NP, FW), _f32),
          jax.ShapeDtypeStruct((NP, 16), _f32),
      ],
  )(msg[0], msg[1], rep8, b1.reshape(1, d), w2, ms16, md16)


def _tail_body(x1_ref, m0_ref, m1_ref, rep_ref, b2_ref,
               wf_ref, uf_ref, bf_ref, wb_ref, ub_ref, bb_ref,
               watt_ref, batt_ref, wout_ref, out_ref):
  x1 = x1_ref[...]
  x2 = _norm(m0_ref[...], m1_ref[...], rep_ref[...]) + b2_ref[...]

  wf = wf_ref[...]
  uf = uf_ref[...]
  bf = bf_ref[...]
  wb = wb_ref[...]
  ub = ub_ref[...]
  bb = bb_ref[...]

  def cell(xt, h, c, w, u, b, first):
    g = jnp.dot(xt, w, preferred_element_type=_f32) + b
    if not first:
      g = g + jnp.dot(h, u, preferred_element_type=_f32)
    i = jax.nn.sigmoid(g[:, 0:HID])
    f = jax.nn.sigmoid(g[:, HID:2 * HID])
    gg = jnp.tanh(g[:, 2 * HID:3 * HID])
    o = jax.nn.sigmoid(g[:, 3 * HID:4 * HID])
    c2 = f * c + i * gg
    return o * jnp.tanh(c2), c2

  zero = jnp.zeros_like(x1)
  hf0, cf0 = cell(x1, zero, zero, wf, uf, bf, True)
  hf1, _ = cell(x2, hf0, cf0, wf, uf, bf, False)
  hb1, cb1 = cell(x2, zero, zero, wb, ub, bb, True)
  hb0, _ = cell(x1, hb1, cb1, wb, ub, bb, False)

  watt = watt_ref[...]
  batt = batt_ref[0, 0]
  a0 = jnp.sum(hf0 * watt[:, 0:HID], axis=1, keepdims=True) + \
       jnp.sum(hb0 * watt[:, HID:2 * HID], axis=1, keepdims=True) + batt
  a1 = jnp.sum(hf1 * watt[:, 0:HID], axis=1, keepdims=True) + \
       jnp.sum(hb1 * watt[:, HID:2 * HID], axis=1, keepdims=True) + batt
  m = jnp.maximum(a0, a1)
  e0 = jnp.exp(a0 - m)
  e1 = jnp.exp(a1 - m)
  zs = e0 + e1
  emb = (e0 / zs) * x1 + (e1 / zs) * x2

  logits = jnp.dot(emb, wout_ref[...], preferred_element_type=_f32)
  lm = jnp.max(logits, axis=1, keepdims=True)
  ls = jnp.log(jnp.sum(jnp.exp(logits - lm), axis=1, keepdims=True))
  out_ref[...] = logits - lm - ls


def _tc_tail(x1, msg2, rep8, b2, wf, uf, bf, wb, ub, bb,
             watt, batt, wout):
  d = HEADS * OUT
  return pl.pallas_call(
      _tail_body,
      grid=(NP // _BLK,),
      in_specs=[
          pl.BlockSpec((_BLK, d), lambda i: (i, 0)),
          pl.BlockSpec((_BLK, FW), lambda i: (i, 0)),
          pl.BlockSpec((_BLK, FW), lambda i: (i, 0)),
          pl.BlockSpec((HEADS, d), lambda i: (0, 0)),
          pl.BlockSpec((1, d), lambda i: (0, 0)),
          pl.BlockSpec((HID, 4 * HID), lambda i: (0, 0)),
          pl.BlockSpec((HID, 4 * HID), lambda i: (0, 0)),
          pl.BlockSpec((1, 4 * HID), lambda i: (0, 0)),
          pl.BlockSpec((HID, 4 * HID), lambda i: (0, 0)),
          pl.BlockSpec((HID, 4 * HID), lambda i: (0, 0)),
          pl.BlockSpec((1, 4 * HID), lambda i: (0, 0)),
          pl.BlockSpec((1, 2 * HID), lambda i: (0, 0)),
          pl.BlockSpec((1, 1), lambda i: (0, 0), memory_space=pltpu.SMEM),
          pl.BlockSpec((HID, NUM_CLASSES), lambda i: (0, 0)),
      ],
      out_specs=pl.BlockSpec((_BLK, NUM_CLASSES), lambda i: (i, 0)),
      out_shape=jax.ShapeDtypeStruct((NP, NUM_CLASSES), _f32),
  )(x1, msg2[0], msg2[1], rep8, b2.reshape(1, d),
    wf, uf, bf.reshape(1, 4 * HID), wb, ub, bb.reshape(1, 4 * HID),
    watt.reshape(1, 2 * HID), batt.reshape(1, 1), wout)


def _att_mat(a):
  m = jnp.zeros((HEADS * OUT, HEADS), _f32)
  m = m.at[jnp.arange(HEADS * OUT), jnp.arange(HEADS * OUT) // OUT].set(
      a.reshape(-1))
  return jnp.concatenate([m, m], axis=1)


def kernel(x, edge_index, W1, a_src1, a_dst1, b1, W2, a_src2, a_dst2, b2,
           W_ih_f, W_hh_f, b_ih_f, b_hh_f, W_ih_b, W_hh_b, b_ih_b, b_hh_b,
           W_att, b_att, W_out):
  loop = jnp.arange(N, dtype=_i32)
  padv = jnp.full((EP - ETOT,), N, _i32)
  src3 = jnp.concatenate([edge_index[0].astype(_i32), loop, padv]
                         ).reshape(NW, ITERS, C)
  dst3 = jnp.concatenate([edge_index[1].astype(_i32), loop, padv]
                         ).reshape(NW, ITERS, C)
  xp = jnp.concatenate([x, jnp.zeros((NP - N, F_IN), _f32)], axis=0)

  ms1, md1 = _att_mat(a_src1), _att_mat(a_dst1)
  ms2, md2 = _att_mat(a_src2), _att_mat(a_dst2)
  rep8 = jnp.kron(jnp.eye(HEADS, dtype=_f32), jnp.ones((1, OUT), _f32))

  xa1, tad1 = _tc_proj(xp, W1, ms1, md1)
  msg1 = _sc_layer(src3, dst3, xa1, tad1)

  x1, xa2, tad2 = _tc_mid(msg1, rep8, b1, W2, ms2, md2)
  msg2 = _sc_layer(src3, dst3, xa2, tad2)

  out = _tc_tail(x1, msg2, rep8, b2,
                 W_ih_f.T, W_hh_f.T, b_ih_f + b_hh_f,
                 W_ih_b.T, W_hh_b.T, b_ih_b + b_hh_b,
                 W_att, b_att, W_out)
  return out[:N]


# speedup vs baseline: 122.0233x; 1.7571x over previous
import jax
import jax.numpy as jnp
from jax import lax
from jax.experimental import pallas as pl
from jax.experimental.pallas import tpu as pltpu
from jax.experimental.pallas import tpu_sc as plsc

N = 10000
E = 320000
HEADS = 8
OUT = 8
HID = 64
NUM_CLASSES = 40
F_IN = 128

NP = 10240
C = 128
ITERS = 82
NW = 32
EP = NW * ITERS * C
ETOT = E + N
ROWS_PER_SUB = NP // 16
FW = 80

_f32 = jnp.float32
_i32 = jnp.int32


def _mesh():
  return plsc.VectorSubcoreMesh(
      core_axis_name="c", subcore_axis_name="s", num_cores=2, num_subcores=16)


def _sc_layer_body(src_hbm, dst_hbm, xa_hbm, tad_hbm,
                   msg_out,
                   sidx2, didx2, b_buf, pp_buf, x_buf, m_buf,
                   sems, macc):
  c = lax.axis_index("c")
  s = lax.axis_index("s")
  wid = s * 2 + c
  lane = lax.iota(_i32, 16)
  lo = lane < 8
  vz = jnp.zeros((16,), _f32)

  pltpu.sync_copy(src_hbm.at[wid], sidx2)
  pltpu.sync_copy(dst_hbm.at[wid], didx2)

  @plsc.parallel_loop(0, C, unroll=8)
  def _zero(r):
    for v in range(FW // 16):
      m_buf[0, r, pl.ds(16 * v, 16)] = vz
  row0 = s * ROWS_PER_SUB
  for k in range(ROWS_PER_SUB // C):
    pltpu.sync_copy(m_buf.at[0], macc.at[pl.ds(row0 + k * C, C)])
  plsc.subcore_barrier()

  idx_hi = jnp.where(lo, lane + 8, lane)
  colvs = [jnp.where(lo, jnp.full((16,), 8 * (w // 4) + 2 * (w % 4), _i32),
                     jnp.full((16,), 8 * (w // 4) + 2 * (w % 4) + 1, _i32))
           for w in range(8)]

  def issue_in(it, slot):
    pltpu.async_copy(xa_hbm.at[sidx2.at[it]], x_buf.at[slot],
                     sems.at[slot, 0])
    pltpu.async_copy(tad_hbm.at[didx2.at[it]], b_buf.at[slot],
                     sems.at[slot, 1])

  def wait_in(it, slot):
    pltpu.make_async_copy(xa_hbm.at[sidx2.at[it]], x_buf.at[slot],
                          sems.at[slot, 0]).wait()
    pltpu.make_async_copy(tad_hbm.at[didx2.at[it]], b_buf.at[slot],
                          sems.at[slot, 1]).wait()

  def issue_scatter(it, slot):
    pltpu.async_copy(m_buf.at[slot], macc.at[didx2.at[it]],
                     sems.at[slot, 2], add=True)

  def wait_scatter(it, slot):
    pltpu.make_async_copy(m_buf.at[slot], macc.at[didx2.at[it]],
                          sems.at[slot, 2]).wait()

  def compute(slot):
    @plsc.parallel_loop(0, C // 2, unroll=4)
    def _pair(j):
      a0 = x_buf[slot, 2 * j, pl.ds(64, 16)]
      a1 = x_buf[slot, 2 * j + 1, pl.ds(64, 16)]
      b0 = b_buf[slot, 2 * j]
      b1 = b_buf[slot, 2 * j + 1]
      al = jnp.where(lo, a0, a1) + jnp.where(lo, b0, b1)
      al = jnp.maximum(al, 0.2 * al)
      p = jnp.exp(al)
      pp_buf[j] = p
      m_buf[slot, 2 * j, pl.ds(64, 16)] = p
      row = jnp.full((16,), j, _i32)
      m_buf[slot, 2 * j + 1, pl.ds(64, 16)] = plsc.load_gather(
          pp_buf, [row, idx_hi])
      for w in range(8):
        pv = plsc.load_gather(pp_buf, [row, colvs[w]])
        e = 2 * j + (w // 4)
        v = w % 4
        m_buf[slot, e, pl.ds(16 * v, 16)] = pv * x_buf[slot, e,
                                                       pl.ds(16 * v, 16)]

  issue_in(0, 0)

  def pair(t, _):
    it0 = t * 2
    it1 = it0 + 1
    issue_in(it1, 1)
    wait_in(it0, 0)

    @pl.when(t > 0)
    def _w0():
      wait_scatter(it0 - 2, 0)

    compute(0)
    issue_scatter(it0, 0)

    @pl.when(t < ITERS // 2 - 1)
    def _p0():
      issue_in(it0 + 2, 0)

    wait_in(it1, 1)

    @pl.when(t > 0)
    def _w1():
      wait_scatter(it1 - 2, 1)

    compute(1)
    issue_scatter(it1, 1)
    return _

  lax.fori_loop(0, ITERS // 2, pair, 0)
  wait_scatter(ITERS - 2, 0)
  wait_scatter(ITERS - 1, 1)

  plsc.subcore_barrier()
  pltpu.sync_copy(macc.at[pl.ds(row0, ROWS_PER_SUB)],
                  msg_out.at[c, pl.ds(row0, ROWS_PER_SUB)])


def _sc_layer(src3, dst3, xa, tad):
  f = pl.kernel(
      _sc_layer_body,
      out_type=jax.ShapeDtypeStruct((2, NP, FW), _f32),
      mesh=_mesh(),
      compiler_params=pltpu.CompilerParams(
          needs_layout_passes=False, use_tc_tiling_on_sc=False),
      scratch_types=[
          pltpu.VMEM((ITERS, C), _i32),
          pltpu.VMEM((ITERS, C), _i32),
          pltpu.VMEM((2, C, 16), _f32),
          pltpu.VMEM((C // 2, 16), _f32),
          pltpu.VMEM((2, C, FW), _f32),
          pltpu.VMEM((2, C, FW), _f32),
          pltpu.SemaphoreType.DMA((2, 3)),
          pltpu.VMEM_SHARED((NP, FW), _f32),
      ],
  )
  return f(src3, dst3, xa, tad)


_BLK = 1024


def _proj_body(x_ref, w_ref, ms_ref, md_ref, xa_ref, tad_ref):
  xl = jnp.dot(x_ref[...], w_ref[...], preferred_element_type=_f32)
  xa_ref[:, 0:HEADS * OUT] = xl
  xa_ref[:, HEADS * OUT:FW] = jnp.dot(xl, ms_ref[...],
                                      preferred_element_type=_f32)
  tad_ref[...] = jnp.dot(xl, md_ref[...], preferred_element_type=_f32)


def _tc_proj(x, w, ms16, md16):
  fin = x.shape[1]
  return pl.pallas_call(
      _proj_body,
      grid=(NP // _BLK,),
      in_specs=[
          pl.BlockSpec((_BLK, fin), lambda i: (i, 0)),
          pl.BlockSpec((fin, HEADS * OUT), lambda i: (0, 0)),
          pl.BlockSpec((HEADS * OUT, 16), lambda i: (0, 0)),
          pl.BlockSpec((HEADS * OUT, 16), lambda i: (0, 0)),
      ],
      out_specs=[
          pl.BlockSpec((_BLK, FW), lambda i: (i, 0)),
          pl.BlockSpec((_BLK, 16), lambda i: (i, 0)),
      ],
      out_shape=[
          jax.ShapeDtypeStruct((NP, FW), _f32),
          jax.ShapeDtypeStruct((NP, 16), _f32),
      ],
  )(x, w, ms16, md16)


def _norm(m0, m1, rep_ref):
  m = m0 + m1
  inv8 = 1.0 / (m[:, HEADS * OUT:HEADS * OUT + HEADS] + 1e-16)
  inv64 = jnp.dot(inv8, rep_ref, preferred_element_type=_f32)
  return m[:, 0:HEADS * OUT] * inv64


def _mid_body(m0_ref, m1_ref, rep_ref, b_ref,
              w_ref, ms_ref, md_ref,
              x1_ref, xa_ref, tad_ref):
  z = _norm(m0_ref[...], m1_ref[...], rep_ref[...]) + b_ref[...]
  x1 = jnp.where(z > 0, z, jnp.exp(jnp.minimum(z, 0.0)) - 1.0)
  x1_ref[...] = x1
  xl = jnp.dot(x1, w_ref[...], preferred_element_type=_f32)
  xa_ref[:, 0:HEADS * OUT] = xl
  xa_ref[:, HEADS * OUT:FW] = jnp.dot(xl, ms_ref[...],
                                      preferred_element_type=_f32)
  tad_ref[...] = jnp.dot(xl, md_ref[...], preferred_element_type=_f32)


def _tc_mid(msg, rep8, b1, w2, ms16, md16):
  d = HEADS * OUT
  return pl.pallas_call(
      _mid_body,
      grid=(NP // _BLK,),
      in_specs=[
          pl.BlockSpec((_BLK, FW), lambda i: (i, 0)),
          pl.BlockSpec((_BLK, FW), lambda i: (i, 0)),
          pl.BlockSpec((HEADS, d), lambda i: (0, 0)),
          pl.BlockSpec((1, d), lambda i: (0, 0)),
          pl.BlockSpec((d, d), lambda i: (0, 0)),
          pl.BlockSpec((d, 16), lambda i: (0, 0)),
          pl.BlockSpec((d, 16), lambda i: (0, 0)),
      ],
      out_specs=[
          pl.BlockSpec((_BLK, d), lambda i: (i, 0)),
          pl.BlockSpec((_BLK, FW), lambda i: (i, 0)),
          pl.BlockSpec((_BLK, 16), lambda i: (i, 0)),
      ],
      out_shape=[
          jax.ShapeDtypeStruct((NP, d), _f32),
          jax.ShapeDtypeStruct((NP, FW), _f32),
          jax.ShapeDtypeStruct((NP, 16), _f32),
      ],
  )(msg[0], msg[1], rep8, b1.reshape(1, d), w2, ms16, md16)


def _tail_body(x1_ref, m0_ref, m1_ref, rep_ref, b2_ref,
               wf_ref, uf_ref, bf_ref, wb_ref, ub_ref, bb_ref,
               watt_ref, batt_ref, wout_ref, out_ref):
  x1 = x1_ref[...]
  x2 = _norm(m0_ref[...], m1_ref[...], rep_ref[...]) + b2_ref[...]

  wf = wf_ref[...]
  uf = uf_ref[...]
  bf = bf_ref[...]
  wb = wb_ref[...]
  ub = ub_ref[...]
  bb = bb_ref[...]

  def cell(xt, h, c, w, u, b, first):
    g = jnp.dot(xt, w, preferred_element_type=_f32) + b
    if not first:
      g = g + jnp.dot(h, u, preferred_element_type=_f32)
    i = jax.nn.sigmoid(g[:, 0:HID])
    f = jax.nn.sigmoid(g[:, HID:2 * HID])
    gg = jnp.tanh(g[:, 2 * HID:3 * HID])
    o = jax.nn.sigmoid(g[:, 3 * HID:4 * HID])
    c2 = f * c + i * gg
    return o * jnp.tanh(c2), c2

  zero = jnp.zeros_like(x1)
  hf0, cf0 = cell(x1, zero, zero, wf, uf, bf, True)
  hf1, _ = cell(x2, hf0, cf0, wf, uf, bf, False)
  hb1, cb1 = cell(x2, zero, zero, wb, ub, bb, True)
  hb0, _ = cell(x1, hb1, cb1, wb, ub, bb, False)

  watt = watt_ref[...]
  batt = batt_ref[0, 0]
  a0 = jnp.sum(hf0 * watt[:, 0:HID], axis=1, keepdims=True) + \
       jnp.sum(hb0 * watt[:, HID:2 * HID], axis=1, keepdims=True) + batt
  a1 = jnp.sum(hf1 * watt[:, 0:HID], axis=1, keepdims=True) + \
       jnp.sum(hb1 * watt[:, HID:2 * HID], axis=1, keepdims=True) + batt
  m = jnp.maximum(a0, a1)
  e0 = jnp.exp(a0 - m)
  e1 = jnp.exp(a1 - m)
  zs = e0 + e1
  emb = (e0 / zs) * x1 + (e1 / zs) * x2

  logits = jnp.dot(emb, wout_ref[...], preferred_element_type=_f32)
  lm = jnp.max(logits, axis=1, keepdims=True)
  ls = jnp.log(jnp.sum(jnp.exp(logits - lm), axis=1, keepdims=True))
  out_ref[...] = logits - lm - ls


def _tc_tail(x1, msg2, rep8, b2, wf, uf, bf, wb, ub, bb,
             watt, batt, wout):
  d = HEADS * OUT
  return pl.pallas_call(
      _tail_body,
      grid=(NP // _BLK,),
      in_specs=[
          pl.BlockSpec((_BLK, d), lambda i: (i, 0)),
          pl.BlockSpec((_BLK, FW), lambda i: (i, 0)),
          pl.BlockSpec((_BLK, FW), lambda i: (i, 0)),
          pl.BlockSpec((HEADS, d), lambda i: (0, 0)),
          pl.BlockSpec((1, d), lambda i: (0, 0)),
          pl.BlockSpec((HID, 4 * HID), lambda i: (0, 0)),
          pl.BlockSpec((HID, 4 * HID), lambda i: (0, 0)),
          pl.BlockSpec((1, 4 * HID), lambda i: (0, 0)),
          pl.BlockSpec((HID, 4 * HID), lambda i: (0, 0)),
          pl.BlockSpec((HID, 4 * HID), lambda i: (0, 0)),
          pl.BlockSpec((1, 4 * HID), lambda i: (0, 0)),
          pl.BlockSpec((1, 2 * HID), lambda i: (0, 0)),
          pl.BlockSpec((1, 1), lambda i: (0, 0), memory_space=pltpu.SMEM),
          pl.BlockSpec((HID, NUM_CLASSES), lambda i: (0, 0)),
      ],
      out_specs=pl.BlockSpec((_BLK, NUM_CLASSES), lambda i: (i, 0)),
      out_shape=jax.ShapeDtypeStruct((NP, NUM_CLASSES), _f32),
  )(x1, msg2[0], msg2[1], rep8, b2.reshape(1, d),
    wf, uf, bf.reshape(1, 4 * HID), wb, ub, bb.reshape(1, 4 * HID),
    watt.reshape(1, 2 * HID), batt.reshape(1, 1), wout)


def _att_mat(a):
  m = jnp.zeros((HEADS * OUT, HEADS), _f32)
  m = m.at[jnp.arange(HEADS * OUT), jnp.arange(HEADS * OUT) // OUT].set(
      a.reshape(-1))
  return jnp.concatenate([m, m], axis=1)


def kernel(x, edge_index, W1, a_src1, a_dst1, b1, W2, a_src2, a_dst2, b2,
           W_ih_f, W_hh_f, b_ih_f, b_hh_f, W_ih_b, W_hh_b, b_ih_b, b_hh_b,
           W_att, b_att, W_out):
  loop = jnp.arange(N, dtype=_i32)
  padv = jnp.full((EP - ETOT,), N, _i32)
  src3 = jnp.concatenate([edge_index[0].astype(_i32), loop, padv]
                         ).reshape(NW, ITERS, C)
  dst3 = jnp.concatenate([edge_index[1].astype(_i32), loop, padv]
                         ).reshape(NW, ITERS, C)
  xp = jnp.concatenate([x, jnp.zeros((NP - N, F_IN), _f32)], axis=0)

  ms1, md1 = _att_mat(a_src1), _att_mat(a_dst1)
  ms2, md2 = _att_mat(a_src2), _att_mat(a_dst2)
  rep8 = jnp.kron(jnp.eye(HEADS, dtype=_f32), jnp.ones((1, OUT), _f32))

  xa1, tad1 = _tc_proj(xp, W1, ms1, md1)
  msg1 = _sc_layer(src3, dst3, xa1, tad1)

  x1, xa2, tad2 = _tc_mid(msg1, rep8, b1, W2, ms2, md2)
  msg2 = _sc_layer(src3, dst3, xa2, tad2)

  out = _tc_tail(x1, msg2, rep8, b2,
                 W_ih_f.T, W_hh_f.T, b_ih_f + b_hh_f,
                 W_ih_b.T, W_hh_b.T, b_ih_b + b_hh_b,
                 W_att, b_att, W_out)
  return out[:N]
